# Initial kernel scaffold; baseline (speedup 1.0000x reference)
import functools

import jax
import jax.numpy as jnp
from jax import lax
from jax.experimental import pallas as pl
from jax.experimental.pallas import tpu as pltpu
from jax.experimental.pallas import tpu_sc as plsc

_N = 10000
_NP = 10240
_D = 128
_E = 320000
_NC = 2
_NS = 16
_NW = _NC * _NS
_EPT = _E // _NW
_CH = 80
_NCHUNK = _EPT // _CH
_RPT = _NP // _NS

_mesh = plsc.VectorSubcoreMesh(core_axis_name="c", subcore_axis_name="s")


def _deg_body(dst_hbm, deg_out, ones_v, dstbuf, zbuf, dbuf, acc):
    c = lax.axis_index("c")
    s = lax.axis_index("s")
    w = c * _NS + s
    z16 = jnp.zeros((16,), jnp.float32)
    o16 = jnp.ones((16,), jnp.float32)
    for r in range(_CH):
        zbuf[r, :] = z16
        ones_v[r, :] = o16
    for i in range(_RPT // _CH):
        pltpu.sync_copy(zbuf, acc.at[pl.ds(s * _RPT + i * _CH, _CH)])
    plsc.subcore_barrier()

    def step(j, carry):
        base = w * _EPT + j * _CH
        pltpu.sync_copy(dst_hbm.at[pl.ds(base, _CH)], dstbuf)
        pltpu.sync_copy(ones_v, acc.at[dstbuf], add=True)
        return carry

    lax.fori_loop(0, _NCHUNK, step, 0)
    plsc.subcore_barrier()
    pltpu.sync_copy(acc.at[pl.ds(s * _RPT, _RPT)], dbuf)
    pltpu.sync_copy(dbuf, deg_out.at[pl.ds(c * _NP + s * _RPT, _RPT)])


_deg_call = pl.kernel(
    _deg_body,
    out_type=jax.ShapeDtypeStruct((_NC * _NP, 16), jnp.float32),
    mesh=_mesh,
    scratch_types=[
        pltpu.VMEM((_CH, 16), jnp.float32),
        pltpu.VMEM((_CH,), jnp.int32),
        pltpu.VMEM((_CH, 16), jnp.float32),
        pltpu.VMEM((_RPT, 16), jnp.float32),
        pltpu.VMEM_SHARED((_NP, 16), jnp.float32),
    ],
)


def _agg_body(src_hbm, dst_hbm, y_hbm, out_hbm, srcbuf, dstbuf, rows, zbuf,
              wbuf, gsem, acc):
    c = lax.axis_index("c")
    s = lax.axis_index("s")
    w = c * _NS + s
    z16 = jnp.zeros((16,), jnp.float32)
    for r in range(32):
        for k in range(8):
            zbuf[r, pl.ds(k * 16, 16)] = z16
    for i in range(_RPT // 32):
        pltpu.sync_copy(zbuf, acc.at[pl.ds(s * _RPT + i * 32, 32)])
    plsc.subcore_barrier()

    def step(j, carry):
        base = w * _EPT + j * _CH
        pltpu.sync_copy(src_hbm.at[pl.ds(base, _CH)], srcbuf)
        pltpu.sync_copy(dst_hbm.at[pl.ds(base, _CH)], dstbuf)
        pltpu.async_copy(y_hbm.at[srcbuf], rows, gsem).wait()
        pltpu.sync_copy(rows, acc.at[dstbuf], add=True)
        return carry

    lax.fori_loop(0, _NCHUNK, step, 0)
    plsc.subcore_barrier()
    for i in range(_RPT // 128):
        pltpu.sync_copy(acc.at[pl.ds(s * _RPT + i * 128, 128)], wbuf)
        pltpu.sync_copy(wbuf, out_hbm.at[pl.ds(c * _NP + s * _RPT + i * 128, 128)])


_agg_call = pl.kernel(
    _agg_body,
    out_type=jax.ShapeDtypeStruct((_NC * _NP, _D), jnp.float32),
    mesh=_mesh,
    scratch_types=[
        pltpu.VMEM((_CH,), jnp.int32),
        pltpu.VMEM((_CH,), jnp.int32),
        pltpu.VMEM((_CH, _D), jnp.float32),
        pltpu.VMEM((32, _D), jnp.float32),
        pltpu.VMEM((128, _D), jnp.float32),
        pltpu.SemaphoreType.DMA,
        pltpu.VMEM_SHARED((_NP, _D), jnp.float32),
    ],
)


_BLK = 512
_GRID = _NP // _BLK


def _mm_body(x_ref, w_ref, d0_ref, d1_ref, y_ref):
    deg = d0_ref[...] + d1_ref[...] + 1.0
    dinv = lax.rsqrt(deg)
    y_ref[...] = jnp.dot(x_ref[...], w_ref[...],
                         preferred_element_type=jnp.float32) * dinv


def _mm(xp, W, d0, d1):
    return pl.pallas_call(
        _mm_body,
        grid=(_GRID,),
        in_specs=[
            pl.BlockSpec((_BLK, _D), lambda i: (i, 0)),
            pl.BlockSpec((_D, _D), lambda i: (0, 0)),
            pl.BlockSpec((_BLK, 1), lambda i: (i, 0)),
            pl.BlockSpec((_BLK, 1), lambda i: (i, 0)),
        ],
        out_specs=pl.BlockSpec((_BLK, _D), lambda i: (i, 0)),
        out_shape=jax.ShapeDtypeStruct((_NP, _D), jnp.float32),
    )(xp, W, d0, d1)


def _comb_core(a0_ref, a1_ref, y_ref, d0_ref, d1_ref, b_ref, g_ref, be_ref):
    deg = d0_ref[...] + d1_ref[...] + 1.0
    dinv = lax.rsqrt(deg)
    t = dinv * (a0_ref[...] + a1_ref[...] + y_ref[...]) + b_ref[...]
    mu = jnp.mean(t, axis=-1, keepdims=True)
    xc = t - mu
    var = jnp.mean(xc * xc, axis=-1, keepdims=True)
    ln = xc * lax.rsqrt(var + 1e-5) * g_ref[...] + be_ref[...]
    return jnp.maximum(ln, 0.0)


def _comb_body(a0_ref, a1_ref, y_ref, d0_ref, d1_ref, b_ref, g_ref, be_ref,
               o_ref):
    o_ref[...] = _comb_core(a0_ref, a1_ref, y_ref, d0_ref, d1_ref, b_ref,
                            g_ref, be_ref)


def _comb_res_body(a0_ref, a1_ref, y_ref, d0_ref, d1_ref, b_ref, g_ref, be_ref,
                   prev_ref, o_ref):
    o_ref[...] = prev_ref[...] + _comb_core(a0_ref, a1_ref, y_ref, d0_ref,
                                            d1_ref, b_ref, g_ref, be_ref)


def _comb(a0, a1, y, d0, d1, b, g, be, prev=None):
    row = lambda i: (i, 0)
    const = lambda i: (0, 0)
    specs = [
        pl.BlockSpec((_BLK, _D), row),
        pl.BlockSpec((_BLK, _D), row),
        pl.BlockSpec((_BLK, _D), row),
        pl.BlockSpec((_BLK, 1), row),
        pl.BlockSpec((_BLK, 1), row),
        pl.BlockSpec((1, _D), const),
        pl.BlockSpec((1, _D), const),
        pl.BlockSpec((1, _D), const),
    ]
    args = [a0, a1, y, d0, d1, b.reshape(1, _D), g.reshape(1, _D),
            be.reshape(1, _D)]
    if prev is None:
        body = _comb_body
    else:
        body = _comb_res_body
        specs.append(pl.BlockSpec((_BLK, _D), row))
        args.append(prev)
    return pl.pallas_call(
        body,
        grid=(_GRID,),
        in_specs=specs,
        out_specs=pl.BlockSpec((_BLK, _D), row),
        out_shape=jax.ShapeDtypeStruct((_NP, _D), jnp.float32),
    )(*args)


def kernel(x, edge_index, W1, b1, g1, be1, W2, b2, g2, be2):
    src = edge_index[0]
    dst = edge_index[1]
    xp = jnp.pad(x, ((0, _NP - _N), (0, 0)))

    degp = _deg_call(dst)
    d0 = degp[:_NP, 0:1]
    d1 = degp[_NP:, 0:1]

    y1 = _mm(xp, W1, d0, d1)
    accp1 = _agg_call(src, dst, y1)
    h1 = _comb(accp1[:_NP], accp1[_NP:], y1, d0, d1, b1, g1, be1)

    y2 = _mm(h1, W2, d0, d1)
    accp2 = _agg_call(src, dst, y2)
    out = _comb(accp2[:_NP], accp2[_NP:], y2, d0, d1, b2, g2, be2, prev=h1)
    return out[:_N]


# speedup vs baseline: 13.0078x; 13.0078x over previous
import jax
import jax.numpy as jnp
from jax import lax
from jax.experimental import pallas as pl
from jax.experimental.pallas import tpu as pltpu
from jax.experimental.pallas import tpu_sc as plsc

_N = 10000
_NP = 10240
_D = 128
_E = 320000
_NC = 2
_NS = 16
_NW = _NC * _NS
_EPT = _E // _NW
_CH = 80
_NCHUNK = _EPT // _CH
_RPT = _NP // _NS
_DR = _NP // _D

_mesh = plsc.VectorSubcoreMesh(core_axis_name="c", subcore_axis_name="s")


def _deg_body(dst_hbm, deg_out, dstv, acc):
    c = lax.axis_index("c")
    s = lax.axis_index("s")
    w = c * _NS + s
    z16 = jnp.zeros((16,), jnp.float32)
    o16 = jnp.ones((16,), jnp.float32)
    for r in range(_DR):
        for k in range(8):
            acc[r, pl.ds(k * 16, 16)] = z16

    def step(j, carry):
        base = w * _EPT + j * _CH
        pltpu.sync_copy(dst_hbm.at[pl.ds(base, _CH)], dstv)
        for g in range(_CH // 16):
            d16 = dstv[pl.ds(g * 16, 16)]
            row = jax.lax.shift_right_logical(d16, 7)
            col = jax.lax.bitwise_and(d16, 127)
            plsc.addupdate_scatter(acc, [row, col], o16)
        return carry

    lax.fori_loop(0, _NCHUNK, step, 0)
    pltpu.sync_copy(acc, deg_out.at[w])


_deg_call = pl.kernel(
    _deg_body,
    out_type=jax.ShapeDtypeStruct((_NW, _DR, _D), jnp.float32),
    mesh=_mesh,
    compiler_params=pltpu.CompilerParams(needs_layout_passes=False),
    scratch_types=[
        pltpu.VMEM((_CH,), jnp.int32),
        pltpu.VMEM((_DR, _D), jnp.float32),
    ],
)


def _agg_body(src_hbm, dst_hbm, y_hbm, out_hbm, srcbuf, dstbuf, rows, zbuf,
              wbuf, gsem, acc):
    c = lax.axis_index("c")
    s = lax.axis_index("s")
    w = c * _NS + s
    z16 = jnp.zeros((16,), jnp.float32)
    for r in range(32):
        for k in range(8):
            zbuf[r, pl.ds(k * 16, 16)] = z16
    for i in range(_RPT // 32):
        pltpu.sync_copy(zbuf, acc.at[pl.ds(s * _RPT + i * 32, 32)])
    plsc.subcore_barrier()

    def step(j, carry):
        base = w * _EPT + j * _CH
        pltpu.sync_copy(src_hbm.at[pl.ds(base, _CH)], srcbuf)
        pltpu.sync_copy(dst_hbm.at[pl.ds(base, _CH)], dstbuf)
        pltpu.async_copy(y_hbm.at[srcbuf], rows, gsem).wait()
        pltpu.sync_copy(rows, acc.at[dstbuf], add=True)
        return carry

    lax.fori_loop(0, _NCHUNK, step, 0)
    plsc.subcore_barrier()
    for i in range(_RPT // 128):
        pltpu.sync_copy(acc.at[pl.ds(s * _RPT + i * 128, 128)], wbuf)
        pltpu.sync_copy(wbuf, out_hbm.at[pl.ds(c * _NP + s * _RPT + i * 128, 128)])


_agg_call = pl.kernel(
    _agg_body,
    out_type=jax.ShapeDtypeStruct((_NC * _NP, _D), jnp.float32),
    mesh=_mesh,
    scratch_types=[
        pltpu.VMEM((_CH,), jnp.int32),
        pltpu.VMEM((_CH,), jnp.int32),
        pltpu.VMEM((_CH, _D), jnp.float32),
        pltpu.VMEM((32, _D), jnp.float32),
        pltpu.VMEM((128, _D), jnp.float32),
        pltpu.SemaphoreType.DMA,
        pltpu.VMEM_SHARED((_NP, _D), jnp.float32),
    ],
)


_BLK = 1024
_GRID = _NP // _BLK
_DPB = _BLK // _D


def _dinv_replicated(deg_ref):
    deg = jnp.sum(deg_ref[...], axis=0) + 1.0
    dinv4 = lax.rsqrt(deg)
    g = jnp.concatenate(
        [jnp.broadcast_to(dinv4[k:k + 1, :], (_D, _D)) for k in range(_DPB)],
        axis=0)
    ri = lax.broadcasted_iota(jnp.int32, (_BLK, _D), 0)
    ci = lax.broadcasted_iota(jnp.int32, (_BLK, _D), 1)
    m = (jnp.bitwise_and(ri, _D - 1) == ci).astype(jnp.float32)
    ones = jnp.ones((_D, _D), jnp.float32)
    return jnp.dot(g * m, ones, preferred_element_type=jnp.float32)


def _mm_body(x_ref, w_ref, deg_ref, y_ref):
    v = _dinv_replicated(deg_ref)
    y_ref[...] = jnp.dot(x_ref[...], w_ref[...],
                         preferred_element_type=jnp.float32) * v


def _mm(xp, W, deg32):
    return pl.pallas_call(
        _mm_body,
        grid=(_GRID,),
        in_specs=[
            pl.BlockSpec((_BLK, _D), lambda i: (i, 0)),
            pl.BlockSpec((_D, _D), lambda i: (0, 0)),
            pl.BlockSpec((_NW, _DPB, _D), lambda i: (0, i, 0)),
        ],
        out_specs=pl.BlockSpec((_BLK, _D), lambda i: (i, 0)),
        out_shape=jax.ShapeDtypeStruct((_NP, _D), jnp.float32),
    )(xp, W, deg32)


def _comb_core(a0_ref, a1_ref, y_ref, deg_ref, b_ref, g_ref, be_ref):
    v = _dinv_replicated(deg_ref)
    t = v * (a0_ref[...] + a1_ref[...] + y_ref[...]) + b_ref[...]
    mu = jnp.mean(t, axis=-1, keepdims=True)
    xc = t - mu
    var = jnp.mean(xc * xc, axis=-1, keepdims=True)
    ln = xc * lax.rsqrt(var + 1e-5) * g_ref[...] + be_ref[...]
    return jnp.maximum(ln, 0.0)


def _comb_body(a0_ref, a1_ref, y_ref, deg_ref, b_ref, g_ref, be_ref, o_ref):
    o_ref[...] = _comb_core(a0_ref, a1_ref, y_ref, deg_ref, b_ref, g_ref,
                            be_ref)


def _comb_res_body(a0_ref, a1_ref, y_ref, deg_ref, b_ref, g_ref, be_ref,
                   prev_ref, o_ref):
    o_ref[...] = prev_ref[...] + _comb_core(a0_ref, a1_ref, y_ref, deg_ref,
                                            b_ref, g_ref, be_ref)


def _comb(a0, a1, y, deg32, b, g, be, prev=None):
    row = lambda i: (i, 0)
    const = lambda i: (0, 0)
    specs = [
        pl.BlockSpec((_BLK, _D), row),
        pl.BlockSpec((_BLK, _D), row),
        pl.BlockSpec((_BLK, _D), row),
        pl.BlockSpec((_NW, _DPB, _D), lambda i: (0, i, 0)),
        pl.BlockSpec((1, _D), const),
        pl.BlockSpec((1, _D), const),
        pl.BlockSpec((1, _D), const),
    ]
    args = [a0, a1, y, deg32, b.reshape(1, _D), g.reshape(1, _D),
            be.reshape(1, _D)]
    if prev is None:
        body = _comb_body
    else:
        body = _comb_res_body
        specs.append(pl.BlockSpec((_BLK, _D), row))
        args.append(prev)
    return pl.pallas_call(
        body,
        grid=(_GRID,),
        in_specs=specs,
        out_specs=pl.BlockSpec((_BLK, _D), row),
        out_shape=jax.ShapeDtypeStruct((_NP, _D), jnp.float32),
    )(*args)


def kernel(x, edge_index, W1, b1, g1, be1, W2, b2, g2, be2):
    src = edge_index[0]
    dst = edge_index[1]
    xp = jnp.pad(x, ((0, _NP - _N), (0, 0)))

    deg32 = _deg_call(dst)

    y1 = _mm(xp, W1, deg32)
    accp1 = _agg_call(src, dst, y1)
    h1 = _comb(accp1[:_NP], accp1[_NP:], y1, deg32, b1, g1, be1)

    y2 = _mm(h1, W2, deg32)
    accp2 = _agg_call(src, dst, y2)
    out = _comb(accp2[:_NP], accp2[_NP:], y2, deg32, b2, g2, be2, prev=h1)
    return out[:_N]
